# Initial kernel scaffold; baseline (speedup 1.0000x reference)
import jax
import jax.numpy as jnp
from jax.experimental import pallas as pl


def kernel(descriptors, pts_src, pts_dst, invis_idx, height, width):
    raise NotImplementedError("write your pallas kernel here")


# speedup vs baseline: 4.0220x; 4.0220x over previous
import jax
import jax.numpy as jnp
from jax import lax
from jax.experimental import pallas as pl
from jax.experimental.pallas import tpu as pltpu

_B, _N, _D = 8, 1024, 256
_R2 = 4.0
_EPS = 1e-8


def _loss_body(fac_ref, invis_ref, ps_ref, pdT_ref, desc_b_ref, desc_a_ref,
               out_ref, acc_ref):
    p = pl.program_id(0)

    @pl.when(p == 0)
    def _init():
        acc_ref[0] = 0.0
        acc_ref[1] = 0.0

    fx = fac_ref[0]
    fy = fac_ref[1]
    ps = ps_ref[0]
    psx = fx * (ps[:, 0:1] + 1.0)
    psy = fy * (ps[:, 1:2] + 1.0)
    pdT = pdT_ref[0, 0]
    pdx = fx * (pdT[0:1, :] + 1.0)
    pdy = fy * (pdT[1:2, :] + 1.0)
    a2 = psx * psx + psy * psy
    b2 = pdx * pdx + pdy * pdy
    ab = psx * pdx + psy * pdy
    d2 = (a2 + b2) - 2.0 * ab

    ri = lax.broadcasted_iota(jnp.int32, (_N, _N), 0)
    ci = lax.broadcasted_iota(jnp.int32, (_N, _N), 1)

    bs = invis_ref[0:1, :]
    bd = invis_ref[1:2, :]
    nn = invis_ref[2:3, :]
    pm = (bs * _B + bd) == p
    niota = lax.broadcasted_iota(jnp.int32, (_N, 1), 0)
    hit = pm & (nn == niota)
    visrow = jnp.logical_not(jnp.any(hit, axis=1, keepdims=True))

    mask = (d2 <= _R2) & (ci > ri) & visrow
    maskf = mask.astype(jnp.float32)
    cnt = jnp.sum(maskf)

    db = desc_b_ref[0]
    da = desc_a_ref[0]
    nb = db / jnp.maximum(jnp.sqrt(jnp.sum(db * db, axis=1, keepdims=True)), _EPS)
    na = da / jnp.maximum(jnp.sqrt(jnp.sum(da * da, axis=1, keepdims=True)), _EPS)
    nb16 = nb.astype(jnp.bfloat16)
    na16 = na.astype(jnp.bfloat16)
    s1 = lax.dot_general(maskf.astype(jnp.bfloat16), na16,
                         (((1,), (0,)), ((), ())),
                         preferred_element_type=jnp.float32)
    dotsum = jnp.sum(nb16.astype(jnp.float32) * s1)

    acc_ref[0] += cnt
    acc_ref[1] += dotsum

    @pl.when(p == _B * _B - 1)
    def _fin():
        out_ref[0, 0] = (acc_ref[0] - acc_ref[1]) / acc_ref[0]


def kernel(descriptors, pts_src, pts_dst, invis_idx, height, width):
    fac = jnp.stack([(width - 1) * 0.5, (height - 1) * 0.5]).astype(jnp.float32)
    pdT = pts_dst.transpose(0, 1, 3, 2)
    invis = invis_idx.astype(jnp.int32)

    out = pl.pallas_call(
        _loss_body,
        grid=(_B * _B,),
        in_specs=[
            pl.BlockSpec(memory_space=pltpu.SMEM),
            pl.BlockSpec((3, 512), lambda p: (0, 0)),
            pl.BlockSpec((1, _N, 2), lambda p: (p % _B, 0, 0)),
            pl.BlockSpec((1, 1, 2, _N), lambda p: (p // _B, p % _B, 0, 0)),
            pl.BlockSpec((1, _N, _D), lambda p: (p % _B, 0, 0)),
            pl.BlockSpec((1, _N, _D), lambda p: (p // _B, 0, 0)),
        ],
        out_specs=pl.BlockSpec(memory_space=pltpu.SMEM),
        out_shape=jax.ShapeDtypeStruct((1, 1), jnp.float32),
        scratch_shapes=[pltpu.SMEM((2,), jnp.float32)],
    )(fac, invis, pts_src, pdT, descriptors, descriptors)
    return out[0, 0]
